# baseline (speedup 1.0000x reference)
import functools

import jax
import jax.numpy as jnp
from jax import lax
from jax.experimental import pallas as pl
from jax.experimental.pallas import tpu as pltpu
from jax.experimental.pallas import tpu_sc as plsc


def _widen_table(table):
    V, D = table.shape
    BLK = 1000

    def body(t_ref, o_ref):
        o_ref[:, :D] = t_ref[...]

    return pl.pallas_call(
        body,
        grid=(V // BLK,),
        in_specs=[pl.BlockSpec((BLK, D), lambda i: (i, 0))],
        out_specs=pl.BlockSpec((BLK, 2 * D), lambda i: (i, 0)),
        out_shape=jax.ShapeDtypeStruct((V, 2 * D), jnp.float32),
    )(table)


def kernel(inputs, table):
    B, S = inputs.shape
    V, D = table.shape
    n_rows = B * S
    table_w = _widen_table(table)
    idx_flat = jnp.bitwise_xor(inputs.reshape(n_rows), 0)

    info = plsc.get_sparse_core_info()
    NC, NS = info.num_cores, info.num_subcores
    NW = NC * NS
    rows_per_w = n_rows // NW
    slabs_per_w = B // NW
    G0 = 128
    G1 = S - G0

    mesh = plsc.VectorSubcoreMesh(core_axis_name="c", subcore_axis_name="s")

    @functools.partial(
        pl.kernel,
        mesh=mesh,
        out_type=jax.ShapeDtypeStruct((B, S, D), jnp.float32),
        scratch_types=[
            pltpu.VMEM((rows_per_w,), jnp.int32),
            pltpu.VMEM((2, S, 128), jnp.float32),
            pltpu.VMEM((2, S, D), jnp.float32),
            pltpu.SemaphoreType.DMA((2,)),
            pltpu.SemaphoreType.DMA((2,)),
        ],
    )
    def gather_kernel(idx_hbm, table_hbm, out_hbm, idx_v, rows_v, rows_c,
                      sem_g, sem_w):
        wid = lax.axis_index("s") * NC + lax.axis_index("c")
        base = wid * rows_per_w
        slab0 = wid * slabs_per_w

        pltpu.sync_copy(idx_hbm.at[pl.ds(base, rows_per_w)], idx_v)

        def fire_gather(i, p):
            off = i * S
            pltpu.async_copy(
                table_hbm.at[idx_v.at[pl.ds(off, G0)]],
                rows_v.at[p, pl.ds(0, G0), :],
                sem_g.at[p],
            )
            pltpu.async_copy(
                table_hbm.at[idx_v.at[pl.ds(off + G0, G1)]],
                rows_v.at[p, pl.ds(G0, G1), :],
                sem_g.at[p],
            )

        def wait_gather(p):
            pltpu.make_async_copy(
                table_hbm.at[idx_v.at[pl.ds(0, G0)]],
                rows_v.at[p, pl.ds(0, G0), :],
                sem_g.at[p],
            ).wait()
            pltpu.make_async_copy(
                table_hbm.at[idx_v.at[pl.ds(0, G1)]],
                rows_v.at[p, pl.ds(G0, G1), :],
                sem_g.at[p],
            ).wait()

        def wait_write(p):
            pltpu.make_async_copy(
                rows_c.at[p], out_hbm.at[slab0], sem_w.at[p]
            ).wait()

        fire_gather(0, 0)

        def body(g, carry):
            for p in (0, 1):
                i = 2 * g + p
                np_ = 1 - p

                @pl.when(i + 1 < slabs_per_w)
                def _():
                    fire_gather(i + 1, np_)

                wait_gather(p)

                @pl.when(i >= 2)
                def _():
                    wait_write(p)

                def compact(q, c2):
                    for u in range(4):
                        r = 4 * q + u
                        for k in range(D // 16):
                            rows_c[p, r, pl.ds(k * 16, 16)] = (
                                rows_v[p, r, pl.ds(k * 16, 16)])
                    return c2

                lax.fori_loop(0, S // 4, compact, 0)
                pltpu.async_copy(
                    rows_c.at[p], out_hbm.at[slab0 + i], sem_w.at[p]
                )
            return carry

        lax.fori_loop(0, slabs_per_w // 2, body, 0)
        wait_write(0)
        wait_write(1)

    return gather_kernel(idx_flat, table_w)


# speedup vs baseline: 1.4663x; 1.4663x over previous
import functools

import jax
import jax.numpy as jnp
from jax import lax
from jax.experimental import pallas as pl
from jax.experimental.pallas import tpu as pltpu
from jax.experimental.pallas import tpu_sc as plsc


def kernel(inputs, table):
    B, S = inputs.shape
    V, D = table.shape
    table_w = jnp.pad(table, ((0, 0), (0, 128 - D)))

    info = plsc.get_sparse_core_info()
    NC, NS = info.num_cores, info.num_subcores
    NW = NC * NS
    slabs_per_w = B // NW
    HS = slabs_per_w // 2
    G0 = 128
    G1 = S - G0

    mesh = plsc.VectorSubcoreMesh(core_axis_name="c", subcore_axis_name="s")

    idx_p = jnp.pad(inputs, ((0, 0), (0, 256 - S)))

    @functools.partial(
        pl.kernel,
        mesh=mesh,
        out_type=jax.ShapeDtypeStruct((B, S, D), jnp.float32),
        scratch_types=[
            pltpu.VMEM((HS, 256), jnp.int32),
            pltpu.VMEM((2, S, 128), jnp.float32),
            pltpu.VMEM((2, S, D), jnp.float32),
            pltpu.SemaphoreType.DMA((2,)),
            pltpu.SemaphoreType.DMA((2,)),
        ],
    )
    def gather_kernel(idx_hbm, table_hbm, out_hbm, idx_v, rows_v,
                      rows_c, sem_g, sem_w):
        wid = lax.axis_index("s") * NC + lax.axis_index("c")
        slab0 = wid * slabs_per_w

        def load_idx(half):
            b0 = slab0 + half * HS
            pltpu.sync_copy(idx_hbm.at[pl.ds(b0, HS), :], idx_v)

        def fire_gather(r, p):
            pltpu.async_copy(
                table_hbm.at[idx_v.at[r, pl.ds(0, G0)]],
                rows_v.at[p, pl.ds(0, G0), :],
                sem_g.at[p],
            )
            pltpu.async_copy(
                table_hbm.at[idx_v.at[r, pl.ds(G0, G1)]],
                rows_v.at[p, pl.ds(G0, G1), :],
                sem_g.at[p],
            )

        def wait_gather(p):
            pltpu.make_async_copy(
                table_hbm.at[idx_v.at[0, pl.ds(0, G0)]],
                rows_v.at[p, pl.ds(0, G0), :],
                sem_g.at[p],
            ).wait()
            pltpu.make_async_copy(
                table_hbm.at[idx_v.at[0, pl.ds(G0, G1)]],
                rows_v.at[p, pl.ds(G0, G1), :],
                sem_g.at[p],
            ).wait()

        def wait_write(p):
            pltpu.make_async_copy(
                rows_c.at[p], out_hbm.at[slab0], sem_w.at[p]
            ).wait()

        for half in (0, 1):
            load_idx(half)
            fire_gather(0, 0)

            def body(g, carry):
                for p in (0, 1):
                    i = 2 * g + p
                    np_ = 1 - p

                    @pl.when(i + 1 < HS)
                    def _():
                        fire_gather(i + 1, np_)

                    wait_gather(p)

                    @pl.when((half > 0) | (i >= 2))
                    def _():
                        wait_write(p)

                    def compact(q, c2):
                        for u in range(4):
                            r = 4 * q + u
                            for k in range(D // 16):
                                rows_c[p, r, pl.ds(k * 16, 16)] = (
                                    rows_v[p, r, pl.ds(k * 16, 16)])
                        return c2

                    lax.fori_loop(0, S // 4, compact, 0)
                    pltpu.async_copy(
                        rows_c.at[p],
                        out_hbm.at[slab0 + half * HS + i],
                        sem_w.at[p],
                    )
                return carry

            lax.fori_loop(0, HS // 2, body, 0)
        wait_write(0)
        wait_write(1)

    return gather_kernel(idx_p, table_w)


# speedup vs baseline: 1.6593x; 1.1316x over previous
import functools

import jax
import jax.numpy as jnp
from jax import lax
from jax.experimental import pallas as pl
from jax.experimental.pallas import tpu as pltpu
from jax.experimental.pallas import tpu_sc as plsc


def kernel(inputs, table):
    B, S = inputs.shape
    V, D = table.shape
    idx_p = jnp.pad(inputs, ((0, 0), (0, 256 - S)))

    info = plsc.get_sparse_core_info()
    NC, NS = info.num_cores, info.num_subcores
    NW = NC * NS
    slabs_per_w = B // NW

    mesh = plsc.VectorSubcoreMesh(core_axis_name="c", subcore_axis_name="s")

    @functools.partial(
        pl.kernel,
        mesh=mesh,
        out_type=jax.ShapeDtypeStruct((B, S, D), jnp.float32),
        scratch_types=[
            pltpu.VMEM((slabs_per_w, 256), jnp.int32),
            pltpu.VMEM((2, S, D), jnp.float32),
            pltpu.SemaphoreType.DMA((2,)),
            pltpu.SemaphoreType.DMA((2,)),
        ],
    )
    def gather_kernel(idx_hbm, table_hbm, out_hbm, idx_v, rows_c,
                      sem_g, sem_w):
        wid = lax.axis_index("s") * NC + lax.axis_index("c")
        slab0 = wid * slabs_per_w

        pltpu.sync_copy(idx_hbm.at[pl.ds(slab0, slabs_per_w), :], idx_v)

        def fire_rows(i, q):
            def grp(g, carry):
                vec = idx_v[i, pl.ds(g * 16, 16)]
                for k in range(16):
                    pltpu.async_copy(
                        table_hbm.at[vec[k]],
                        rows_c.at[q, g * 16 + k],
                        sem_g.at[q],
                    )
                return carry

            lax.fori_loop(0, S // 16, grp, 0)
            vec = idx_v[i, pl.ds((S // 16) * 16, 16)]
            for k in range(S - (S // 16) * 16):
                pltpu.async_copy(
                    table_hbm.at[vec[k]],
                    rows_c.at[q, (S // 16) * 16 + k],
                    sem_g.at[q],
                )

        def drain_rows(q):
            def row(j, carry):
                pltpu.make_async_copy(
                    table_hbm.at[0], rows_c.at[q, 0], sem_g.at[q]
                ).wait()
                return carry

            lax.fori_loop(0, S, row, 0)

        def wait_write(q):
            pltpu.make_async_copy(
                rows_c.at[q], out_hbm.at[slab0], sem_w.at[q]
            ).wait()

        fire_rows(0, 0)

        def body(g, carry):
            for p in (0, 1):
                i = 2 * g + p
                np_ = 1 - p

                @pl.when(i >= 1)
                def _():
                    wait_write(np_)

                @pl.when(i + 1 < slabs_per_w)
                def _():
                    fire_rows(i + 1, np_)

                drain_rows(p)
                pltpu.async_copy(
                    rows_c.at[p], out_hbm.at[slab0 + i], sem_w.at[p]
                )
            return carry

        lax.fori_loop(0, slabs_per_w // 2, body, 0)
        wait_write(1)

    return gather_kernel(idx_p, table)
